# Initial kernel scaffold; baseline (speedup 1.0000x reference)
import jax
import jax.numpy as jnp
from jax.experimental import pallas as pl


def kernel(unique_params, index_map):
    raise NotImplementedError("write your pallas kernel here")


# speedup vs baseline: 1.2008x; 1.2008x over previous
import functools

import jax
import jax.numpy as jnp
from jax import lax
from jax.experimental import pallas as pl
from jax.experimental.pallas import tpu as pltpu
from jax.experimental.pallas import tpu_sc as plsc

H = W = 14
HW = H * W
NROWS = HW * HW
CHUNK = 112
NCHUNK = NROWS // CHUNK


def kernel(unique_params, index_map):
    info = plsc.get_sparse_core_info()
    nc, ns = info.num_cores, info.num_subcores
    nw = nc * ns
    trips = -(-NCHUNK // nw)

    mesh = plsc.VectorSubcoreMesh(core_axis_name="c", subcore_axis_name="s")

    @functools.partial(
        pl.kernel,
        mesh=mesh,
        out_type=jax.ShapeDtypeStruct((NROWS, 1024), jnp.float32),
        scratch_types=[
            pltpu.VMEM((CHUNK,), jnp.int32),
            pltpu.VMEM((CHUNK, 1024), jnp.float32),
            pltpu.SemaphoreType.DMA,
        ],
    )
    def gather_rows(table_hbm, idx_hbm, out_hbm, idx_v, rows_v, sem):
        wid = lax.axis_index("s") * nc + lax.axis_index("c")

        def body(t, carry):
            c = wid + nw * t

            @pl.when(c < NCHUNK)
            def _():
                base = c * CHUNK
                pltpu.sync_copy(idx_hbm.at[pl.ds(base, CHUNK)], idx_v)
                pltpu.async_copy(table_hbm.at[idx_v], rows_v, sem).wait()
                pltpu.sync_copy(rows_v, out_hbm.at[pl.ds(base, CHUNK)])

            return carry

        lax.fori_loop(0, trips, body, None)

    out = gather_rows(unique_params.reshape(729, 1024), index_map.reshape(NROWS))
    return out.reshape(HW, HW, 32, 32)


# speedup vs baseline: 1.2093x; 1.0071x over previous
import functools

import jax
import jax.numpy as jnp
from jax import lax
from jax.experimental import pallas as pl
from jax.experimental.pallas import tpu as pltpu
from jax.experimental.pallas import tpu_sc as plsc

H = W = 14
HW = H * W
NROWS = HW * HW
CHUNK = 56
NCHUNK = NROWS // CHUNK


def kernel(unique_params, index_map):
    info = plsc.get_sparse_core_info()
    nc, ns = info.num_cores, info.num_subcores
    nw = nc * ns
    trips = -(-NCHUNK // nw)
    assert trips % 2 == 0

    mesh = plsc.VectorSubcoreMesh(core_axis_name="c", subcore_axis_name="s")

    @functools.partial(
        pl.kernel,
        mesh=mesh,
        out_type=jax.ShapeDtypeStruct((NROWS, 1024), jnp.float32),
        scratch_types=[
            pltpu.VMEM((CHUNK,), jnp.int32),
            pltpu.VMEM((CHUNK,), jnp.int32),
            pltpu.VMEM((CHUNK, 1024), jnp.float32),
            pltpu.VMEM((CHUNK, 1024), jnp.float32),
            pltpu.SemaphoreType.DMA,
            pltpu.SemaphoreType.DMA,
        ],
    )
    def gather_rows(table_hbm, idx_hbm, out_hbm,
                    idx_v0, idx_v1, rows_v0, rows_v1, sem0, sem1):
        wid = lax.axis_index("s") * nc + lax.axis_index("c")
        idx_v = (idx_v0, idx_v1)
        rows_v = (rows_v0, rows_v1)
        sem = (sem0, sem1)

        def start(t, b):
            c = wid + nw * t

            @pl.when(c < NCHUNK)
            def _():
                pltpu.sync_copy(idx_hbm.at[pl.ds(c * CHUNK, CHUNK)], idx_v[b])
                pltpu.async_copy(table_hbm.at[idx_v[b]], rows_v[b], sem[b])

        def finish(t, b):
            c = wid + nw * t

            @pl.when(c < NCHUNK)
            def _():
                pltpu.make_async_copy(table_hbm.at[idx_v[b]],
                                      rows_v[b], sem[b]).wait()
                pltpu.sync_copy(rows_v[b], out_hbm.at[pl.ds(c * CHUNK, CHUNK)])

        start(0, 0)

        def body(u, carry):
            t0 = 2 * u
            start(t0 + 1, 1)
            finish(t0, 0)
            start(t0 + 2, 0)
            finish(t0 + 1, 1)
            return carry

        lax.fori_loop(0, trips // 2, body, None)

    out = gather_rows(unique_params.reshape(729, 1024), index_map.reshape(NROWS))
    return out.reshape(HW, HW, 32, 32)
